# Initial kernel scaffold; baseline (speedup 1.0000x reference)
import jax
import jax.numpy as jnp
from jax.experimental import pallas as pl


def kernel(x, W1_conv, b1_conv, W1_nn, b1_nn, W1_pw, b1_pw, W2_conv, b2_conv, W2_nn, b2_nn, W2_pw, b2_pw, Wfc1, bfc1, Wfc2, bfc2):
    raise NotImplementedError("write your pallas kernel here")


# speedup vs baseline: 8.1957x; 8.1957x over previous
import jax
import jax.numpy as jnp
from jax import lax
from jax.experimental import pallas as pl
from jax.experimental.pallas import tpu as pltpu


def _mm_cmaj(A, P, bias, tn=8192):
    K, O = A.shape
    _, NC = P.shape
    grid = NC // tn

    def body(a_ref, p_ref, b_ref, o_ref):
        o_ref[...] = lax.dot_general(
            a_ref[...], p_ref[...], (((0,), (0,)), ((), ())),
            preferred_element_type=jnp.float32) + b_ref[...]

    return pl.pallas_call(
        body,
        grid=(grid,),
        in_specs=[
            pl.BlockSpec((K, O), lambda i: (0, 0)),
            pl.BlockSpec((K, tn), lambda i: (0, i)),
            pl.BlockSpec((O, 1), lambda i: (0, 0)),
        ],
        out_specs=pl.BlockSpec((O, tn), lambda i: (0, i)),
        out_shape=jax.ShapeDtypeStruct((O, NC), jnp.float32),
    )(A, P, bias)


def _conv3x3_cmaj(xc, W, b, tn=8192):
    C, B, H, Wd = xc.shape
    O = W.shape[0]
    xp = jnp.pad(xc, ((0, 0), (0, 0), (1, 1), (1, 1)))
    P = jnp.stack([xp[:, :, dy:dy + H, dx:dx + Wd]
                   for dy in range(3) for dx in range(3)], 0)
    P = P.reshape(9 * C, B * H * Wd)
    Wf = W.transpose(2, 3, 1, 0).reshape(9 * C, O)
    return _mm_cmaj(Wf, P, b.reshape(O, 1), tn=tn)


def _convnn(yt, Wf, bias, K=9):
    B, N, C = yt.shape
    KC, Oss = Wf.shape

    def body(yt_ref, w_ref, b_ref, o_ref):
        ytb = yt_ref[0]
        sim = lax.dot_general(ytb, ytb, (((1,), (1,)), ((), ())),
                              preferred_element_type=jnp.float32)
        col = lax.broadcasted_iota(jnp.int32, (N, N), 1)
        big = jnp.int32(2**30)
        gs = []
        for k in range(K):
            m = jnp.max(sim, axis=1, keepdims=True)
            cand = jnp.where(sim == m, col, big)
            amin = jnp.min(cand, axis=1, keepdims=True)
            oh = col == amin
            gs.append(lax.dot_general(oh.astype(jnp.float32), ytb,
                                      (((1,), (0,)), ((), ())),
                                      preferred_element_type=jnp.float32))
            if k < K - 1:
                sim = jnp.where(oh, -jnp.inf, sim)
        G = jnp.concatenate(gs, axis=1)
        o_ref[0] = lax.dot_general(G, w_ref[...], (((1,), (0,)), ((), ())),
                                   preferred_element_type=jnp.float32) + b_ref[...]

    return pl.pallas_call(
        body,
        grid=(B,),
        in_specs=[
            pl.BlockSpec((1, N, C), lambda i: (i, 0, 0)),
            pl.BlockSpec((KC, Oss), lambda i: (0, 0)),
            pl.BlockSpec((1, Oss), lambda i: (0, 0)),
        ],
        out_specs=pl.BlockSpec((1, N, Oss), lambda i: (i, 0, 0)),
        out_shape=jax.ShapeDtypeStruct((B, N, Oss), jnp.float32),
    )(yt, Wf, bias)


def _fc_head(h, Wfc1, bfc1, Wfc2, bfc2, tk=2048):
    B, F = h.shape
    H1 = Wfc1.shape[0]
    O = Wfc2.shape[0]
    steps = F // tk

    def body(h_ref, w1_ref, b1_ref, w2_ref, b2_ref, o_ref, acc):
        i = pl.program_id(0)

        @pl.when(i == 0)
        def _init():
            acc[...] = jnp.zeros_like(acc)

        acc[...] += lax.dot_general(h_ref[...], w1_ref[...],
                                    (((1,), (1,)), ((), ())),
                                    preferred_element_type=jnp.float32)

        @pl.when(i == steps - 1)
        def _fin():
            r = jnp.maximum(acc[...] + b1_ref[...], 0.0)
            o_ref[...] = lax.dot_general(r, w2_ref[...],
                                         (((1,), (1,)), ((), ())),
                                         preferred_element_type=jnp.float32) + b2_ref[...]

    return pl.pallas_call(
        body,
        grid=(steps,),
        in_specs=[
            pl.BlockSpec((B, tk), lambda i: (0, i)),
            pl.BlockSpec((H1, tk), lambda i: (0, i)),
            pl.BlockSpec((1, H1), lambda i: (0, 0)),
            pl.BlockSpec((O, H1), lambda i: (0, 0)),
            pl.BlockSpec((1, O), lambda i: (0, 0)),
        ],
        out_specs=pl.BlockSpec((B, O), lambda i: (0, 0)),
        out_shape=jax.ShapeDtypeStruct((B, O), jnp.float32),
        scratch_shapes=[pltpu.VMEM((B, H1), jnp.float32)],
    )(h, Wfc1, bfc1.reshape(1, H1), Wfc2, bfc2.reshape(1, O))


def _unshuffle_cmaj(xc, s=2):
    C, B, H, W = xc.shape
    y = xc.reshape(C, B, H // s, s, W // s, s).transpose(0, 3, 5, 1, 2, 4)
    y = y.reshape(C * s * s, B, (H // s) * (W // s))
    return y.transpose(1, 2, 0)


def _shuffle_cmaj(o2, Hh, Wh, s=2):
    B, N, Oss = o2.shape
    O = Oss // (s * s)
    z = o2.reshape(B, Hh, Wh, O, s, s).transpose(3, 0, 1, 4, 2, 5)
    return z.reshape(O, B * Hh * s * Wh * s)


def _branch_layer(xc, Wc, bc, Wn, bn, Wp, bp, K=9, s=2):
    C, B, H, W = xc.shape
    b1 = _conv3x3_cmaj(xc, Wc, bc)
    yt = _unshuffle_cmaj(xc, s)
    Oss, Css, _ = Wn.shape
    Wnf = Wn.transpose(2, 1, 0).reshape(K * Css, Oss)
    o2 = _convnn(yt, Wnf, bn.reshape(1, Oss), K=K)
    b2 = _shuffle_cmaj(o2, H // s, W // s, s)
    cat = jnp.concatenate([b1, b2], axis=0)
    O2 = Wp.shape[0]
    Wpm = Wp.reshape(O2, Wp.shape[1]).transpose(1, 0)
    return _mm_cmaj(Wpm, cat, bp.reshape(O2, 1))


def kernel(x, W1_conv, b1_conv, W1_nn, b1_nn, W1_pw, b1_pw,
           W2_conv, b2_conv, W2_nn, b2_nn, W2_pw, b2_pw,
           Wfc1, bfc1, Wfc2, bfc2):
    B = x.shape[0]
    xc = x.transpose(1, 0, 2, 3)
    h1 = _branch_layer(xc, W1_conv, b1_conv, W1_nn, b1_nn, W1_pw, b1_pw)
    h1 = h1.reshape(16, B, 32, 32)
    h2 = _branch_layer(h1, W2_conv, b2_conv, W2_nn, b2_nn, W2_pw, b2_pw)
    h = h2.reshape(32, B, 1024).transpose(1, 0, 2).reshape(B, 32 * 1024)
    return _fc_head(h, Wfc1, bfc1, Wfc2, bfc2)
